# baseline (speedup 1.0000x reference)
import dataclasses
import functools

import jax
import jax.numpy as jnp
from jax import lax
from jax.experimental import pallas as pl
from jax.experimental.pallas import tpu as pltpu
from jax.experimental.pallas import tpu_sc as plsc

_N = 10000
_NP = 10240
_E = 320000
_NC, _NS, _L = 2, 16, 16
_NW = _NC * _NS

_K = 80
_EPT_COL = 20160
_EPT_EDGE = 10240
_EPT_DEG = _E // _NW
_RPT = _NP // _NS
_ZR = 64

_R = 2048


def _sc_compiler_params():
    cp = pltpu.CompilerParams()
    if "needs_layout_passes" in pltpu.CompilerParams.__dataclass_fields__:
        cp = dataclasses.replace(cp, needs_layout_passes=False)
    return cp


def _vmesh():
    return plsc.VectorSubcoreMesh(core_axis_name="c", subcore_axis_name="s")


def _sc_degree(dst, w):
    @functools.partial(
        pl.kernel,
        out_type=jax.ShapeDtypeStruct((_NW, _NP), jnp.float32),
        mesh=_vmesh(),
        compiler_params=_sc_compiler_params(),
        scratch_types=[
            pltpu.VMEM((_NP,), jnp.float32),
            pltpu.VMEM((_EPT_DEG,), jnp.int32),
            pltpu.VMEM((_EPT_DEG,), jnp.float32),
        ],
    )
    def k(dst_hbm, w_hbm, out_hbm, hist, idxb, wb):
        c = lax.axis_index("c")
        s = lax.axis_index("s")
        wid = c * _NS + s
        zeros = jnp.zeros((_L,), jnp.float32)

        @pl.loop(0, _NP // _L)
        def _(i):
            hist[pl.ds(i * _L, _L)] = zeros

        base = wid * _EPT_DEG
        pltpu.sync_copy(dst_hbm.at[pl.ds(base, _EPT_DEG)], idxb)
        pltpu.sync_copy(w_hbm.at[pl.ds(base, _EPT_DEG)], wb)

        @pl.loop(0, _EPT_DEG // _L)
        def _(i):
            sl = pl.ds(i * _L, _L)
            plsc.addupdate_scatter(hist, [idxb[sl]], wb[sl])

        pltpu.sync_copy(hist, out_hbm.at[wid])

    return k(dst, w)


_SLOTS = 4


def _make_sc_agg(col_split):
    ept = _EPT_COL if col_split else _EPT_EDGE
    n_ch = ept // _K
    n_blk = n_ch // _SLOTS

    @functools.partial(
        pl.kernel,
        out_type=jax.ShapeDtypeStruct((2 * _NP, 128), jnp.float32),
        mesh=_vmesh(),
        compiler_params=_sc_compiler_params(),
        scratch_types=[
            pltpu.VMEM_SHARED((_NP, 128), jnp.float32),
            pltpu.VMEM((_SLOTS, _K), jnp.int32),
            pltpu.VMEM((_SLOTS, _K), jnp.int32),
            pltpu.VMEM((_SLOTS, _K), jnp.float32),
            pltpu.VMEM((_SLOTS * _K, 128), jnp.float32),
            pltpu.SemaphoreType.DMA((_SLOTS,)),
            pltpu.SemaphoreType.DMA((_SLOTS,)),
            pltpu.SemaphoreType.DMA((_SLOTS,)),
        ],
    )
    def k(h_hbm, src_hbm, dst_hbm, w_hbm, out_hbm,
          acc, sidx, didx, wb, rows, si, sg, ss):
        c = lax.axis_index("c")
        s = lax.axis_index("s")
        zeros = jnp.zeros((_L,), jnp.float32)

        if col_split:
            ebase = s * _EPT_COL
            coff = c * _NP
        else:
            ebase = (c * _NS + s) * _EPT_EDGE
            coff = None

        def issue_idx(ci, sl):
            o = ebase + ci * _K
            pltpu.async_copy(src_hbm.at[pl.ds(o, _K)], sidx.at[sl], si.at[sl])
            pltpu.async_copy(dst_hbm.at[pl.ds(o, _K)], didx.at[sl], si.at[sl])
            pltpu.async_copy(w_hbm.at[pl.ds(o, _K)], wb.at[sl], si.at[sl])

        def start_gather(ci, sl):
            pltpu.make_async_copy(src_hbm.at[pl.ds(0, _K)], sidx.at[sl], si.at[sl]).wait()
            pltpu.make_async_copy(dst_hbm.at[pl.ds(0, _K)], didx.at[sl], si.at[sl]).wait()
            pltpu.make_async_copy(w_hbm.at[pl.ds(0, _K)], wb.at[sl], si.at[sl]).wait()
            if col_split:
                for j in range(_K // _L):
                    sv = sidx[sl, pl.ds(j * _L, _L)]
                    sidx[sl, pl.ds(j * _L, _L)] = sv + coff
            pltpu.async_copy(h_hbm.at[sidx.at[sl]],
                             rows.at[pl.ds(sl * _K, _K)], sg.at[sl])

        def finish(ci, sl):
            pltpu.make_async_copy(h_hbm.at[sidx.at[sl]],
                                  rows.at[pl.ds(sl * _K, _K)], sg.at[sl]).wait()

            @pl.loop(0, _K)
            def _(i):
                wv = plsc.load_gather(wb.at[sl], [jnp.full((_L,), i, jnp.int32)])
                r = sl * _K + i
                for j in range(128 // _L):
                    rows[r, pl.ds(j * _L, _L)] = rows[r, pl.ds(j * _L, _L)] * wv

            pltpu.async_copy(rows.at[pl.ds(sl * _K, _K)],
                             acc.at[didx.at[sl]], ss.at[sl], add=True)

        def retire(sl):
            pltpu.make_async_copy(rows.at[pl.ds(sl * _K, _K)],
                                  acc.at[didx.at[sl]], ss.at[sl]).wait()

        issue_idx(0, 0)
        issue_idx(1, 1)

        @pl.loop(0, _ZR)
        def _(r):
            for j in range(128 // _L):
                rows[r, pl.ds(j * _L, _L)] = zeros

        rbase = s * _RPT
        for t in range(_RPT // _ZR):
            pltpu.sync_copy(rows.at[pl.ds(0, _ZR)],
                            acc.at[pl.ds(rbase + t * _ZR, _ZR)])
        plsc.subcore_barrier()

        start_gather(0, 0)
        for cp in range(_SLOTS):
            if cp >= 3:
                retire((cp + 2) % _SLOTS)
            issue_idx(cp + 2, (cp + 2) % _SLOTS)
            start_gather(cp + 1, (cp + 1) % _SLOTS)
            finish(cp, cp)

        @pl.loop(1, n_blk - 1)
        def _(blk):
            cb = blk * _SLOTS
            for r in range(_SLOTS):
                retire((r + 2) % _SLOTS)
                issue_idx(cb + r + 2, (r + 2) % _SLOTS)
                start_gather(cb + r + 1, (r + 1) % _SLOTS)
                finish(cb + r, r)

        for cp in range(n_ch - _SLOTS, n_ch):
            r = cp % _SLOTS
            retire((r + 2) % _SLOTS)
            if cp + 2 < n_ch:
                issue_idx(cp + 2, (r + 2) % _SLOTS)
            if cp + 1 < n_ch:
                start_gather(cp + 1, (r + 1) % _SLOTS)
            finish(cp, r)
        for sl in range(2, _SLOTS):
            retire(sl)

        plsc.subcore_barrier()
        coff_out = c * _NP
        for t in range(_RPT // _ZR):
            rr = rbase + t * _ZR
            pltpu.sync_copy(acc.at[pl.ds(rr, _ZR)],
                            out_hbm.at[pl.ds(coff_out + rr, _ZR)])

    return k


def _sc_agg_cols(h_flat, src, dst, w):
    return _make_sc_agg(True)(h_flat, src, dst, w)


def _sc_agg_edges(h3, src, dst, w):
    return _make_sc_agg(False)(h3, src, dst, w)


def _tc_dinv(degp):
    def body(deg_ref, o_ref):
        deg = jnp.sum(deg_ref[...], axis=0) + 1.0
        o_ref[...] = jnp.where(deg > 0.0,
                               lax.rsqrt(jnp.maximum(deg, 1e-12)), 0.0)

    return pl.pallas_call(
        body,
        out_shape=jax.ShapeDtypeStruct((128, 80), jnp.float32),
    )(degp)


def _tc_matmul(x, W):
    M, K = x.shape
    Nw = W.shape[1]

    def body(x_ref, w_ref, o_ref):
        o_ref[...] = jnp.dot(x_ref[...], w_ref[...],
                             preferred_element_type=jnp.float32)

    return pl.pallas_call(
        body,
        grid=(M // _R,),
        in_specs=[
            pl.BlockSpec((_R, K), lambda i: (i, 0)),
            pl.BlockSpec((K, Nw), lambda i: (0, 0)),
        ],
        out_specs=pl.BlockSpec((_R, Nw), lambda i: (i, 0)),
        out_shape=jax.ShapeDtypeStruct((M, Nw), jnp.float32),
    )(x, W)


def _tc_scale_split(g, dinv):
    def body(g_ref, dinv_ref, o_ref):
        h = g_ref[...] * dinv_ref[...]
        o_ref[0] = h[:, :128]
        o_ref[1] = h[:, 128:]

    return pl.pallas_call(
        body,
        grid=(_NP // _R,),
        in_specs=[
            pl.BlockSpec((_R, 256), lambda i: (i, 0)),
            pl.BlockSpec((_R, 1), lambda i: (i, 0)),
        ],
        out_specs=pl.BlockSpec((2, _R, 128), lambda i: (0, i, 0)),
        out_shape=jax.ShapeDtypeStruct((2, _NP, 128), jnp.float32),
    )(g, dinv)


def _tc_layer(acc, hs, dinv, b, W):
    Nw = W.shape[1]

    def body(acc_ref, hs_ref, dinv_ref, b_ref, w_ref, o_ref):
        dinv = dinv_ref[...]
        bb = b_ref[...]
        t0 = jnp.maximum(dinv * (acc_ref[0] + hs_ref[0]) + bb[:, :128], 0.0)
        t1 = jnp.maximum(dinv * (acc_ref[1] + hs_ref[1]) + bb[:, 128:], 0.0)
        t = jnp.concatenate([t0, t1], axis=1)
        g = jnp.dot(t, w_ref[...], preferred_element_type=jnp.float32)
        h = g * dinv
        if Nw == 256:
            o_ref[0] = h[:, :128]
            o_ref[1] = h[:, 128:]
        else:
            o_ref[...] = h

    if Nw == 256:
        out_spec = pl.BlockSpec((2, _R, 128), lambda i: (0, i, 0))
        out_shape = jax.ShapeDtypeStruct((2, _NP, 128), jnp.float32)
    else:
        out_spec = pl.BlockSpec((_R, Nw), lambda i: (i, 0))
        out_shape = jax.ShapeDtypeStruct((_NP, Nw), jnp.float32)

    return pl.pallas_call(
        body,
        grid=(_NP // _R,),
        in_specs=[
            pl.BlockSpec((2, _R, 128), lambda i: (0, i, 0)),
            pl.BlockSpec((2, _R, 128), lambda i: (0, i, 0)),
            pl.BlockSpec((_R, 1), lambda i: (i, 0)),
            pl.BlockSpec((1, 256), lambda i: (0, 0)),
            pl.BlockSpec(W.shape, lambda i: (0, 0)),
        ],
        out_specs=out_spec,
        out_shape=out_shape,
    )(acc, hs, dinv, b, W)


def _tc_final(accp, h3, dinv, b3):
    def body(acc_ref, h_ref, dinv_ref, b_ref, o_ref):
        sagg = acc_ref[0] + acc_ref[1] + h_ref[...]
        z = (dinv_ref[...] * sagg + b_ref[...])[:, :40]
        m = jnp.max(z, axis=1, keepdims=True)
        ze = z - m
        lse = jnp.log(jnp.sum(jnp.exp(ze), axis=1, keepdims=True))
        o_ref[...] = ze - lse

    return pl.pallas_call(
        body,
        grid=(_NP // _R,),
        in_specs=[
            pl.BlockSpec((2, _R, 128), lambda i: (0, i, 0)),
            pl.BlockSpec((_R, 128), lambda i: (i, 0)),
            pl.BlockSpec((_R, 1), lambda i: (i, 0)),
            pl.BlockSpec((1, 128), lambda i: (0, 0)),
        ],
        out_specs=pl.BlockSpec((_R, 40), lambda i: (i, 0)),
        out_shape=jax.ShapeDtypeStruct((_NP, 40), jnp.float32),
    )(accp, h3, dinv, b3)


def _pad_edges(a, tiles, real, padded):
    return jnp.pad(a.reshape(tiles, real),
                   ((0, 0), (0, padded - real))).reshape(-1)


def kernel(x, edge_index, edge_attr, W1, b1, W2, b2, W3, b3):
    src = edge_index[0].astype(jnp.int32)
    dst = edge_index[1].astype(jnp.int32)
    w = edge_attr.astype(jnp.float32)
    xp = jnp.pad(x, ((0, _NP - _N), (0, 0)))

    src_c = _pad_edges(src, _NS, _E // _NS, _EPT_COL)
    dst_c = _pad_edges(dst, _NS, _E // _NS, _EPT_COL)
    w_c = _pad_edges(w, _NS, _E // _NS, _EPT_COL)
    src_e = _pad_edges(src, _NW, _E // _NW, _EPT_EDGE)
    dst_e = _pad_edges(dst, _NW, _E // _NW, _EPT_EDGE)
    w_e = _pad_edges(w, _NW, _E // _NW, _EPT_EDGE)

    degp = _sc_degree(dst, w)
    g1 = _tc_matmul(xp, W1)
    dinv = _tc_dinv(degp.reshape(_NW, 128, 80)).reshape(_NP, 1)

    h1 = _tc_scale_split(g1, dinv)
    acc1 = _sc_agg_cols(h1.reshape(2 * _NP, 128), src_c, dst_c, w_c)
    h2 = _tc_layer(acc1.reshape(2, _NP, 128), h1, dinv,
                   b1.reshape(1, 256), W2)

    acc2 = _sc_agg_cols(h2.reshape(2 * _NP, 128), src_c, dst_c, w_c)
    W3p = jnp.pad(W3, ((0, 0), (0, 88)))
    h3 = _tc_layer(acc2.reshape(2, _NP, 128), h2, dinv,
                   b2.reshape(1, 256), W3p)

    acc3 = _sc_agg_edges(h3, src_e, dst_e, w_e)
    b3p = jnp.pad(b3, (0, 88)).reshape(1, 128)
    out = _tc_final(acc3.reshape(2, _NP, 128), h3, dinv, b3p)
    return out[:_N]


# speedup vs baseline: 1.3048x; 1.3048x over previous
import dataclasses
import functools

import jax
import jax.numpy as jnp
from jax import lax
from jax.experimental import pallas as pl
from jax.experimental.pallas import tpu as pltpu
from jax.experimental.pallas import tpu_sc as plsc

_N = 10000
_NP = 10240
_E = 320000
_NC, _NS, _L = 2, 16, 16
_NW = _NC * _NS

_K = 40
_EPT_COL = _E // _NS
_EPT_EDGE = _E // _NW
_RPT = _NP // _NS
_ZR = 64

_R = 2048


def _sc_compiler_params():
    cp = pltpu.CompilerParams()
    if "needs_layout_passes" in pltpu.CompilerParams.__dataclass_fields__:
        cp = dataclasses.replace(cp, needs_layout_passes=False)
    return cp


def _vmesh():
    return plsc.VectorSubcoreMesh(core_axis_name="c", subcore_axis_name="s")


def _sc_degree(dst, w):
    @functools.partial(
        pl.kernel,
        out_type=jax.ShapeDtypeStruct((_NW, _NP), jnp.float32),
        mesh=_vmesh(),
        compiler_params=_sc_compiler_params(),
        scratch_types=[
            pltpu.VMEM((_NP,), jnp.float32),
            pltpu.VMEM((_EPT_EDGE,), jnp.int32),
            pltpu.VMEM((_EPT_EDGE,), jnp.float32),
        ],
    )
    def k(dst_hbm, w_hbm, out_hbm, hist, idxb, wb):
        c = lax.axis_index("c")
        s = lax.axis_index("s")
        wid = c * _NS + s
        zeros = jnp.zeros((_L,), jnp.float32)

        @pl.loop(0, _NP // _L)
        def _(i):
            hist[pl.ds(i * _L, _L)] = zeros

        base = wid * _EPT_EDGE
        pltpu.sync_copy(dst_hbm.at[pl.ds(base, _EPT_EDGE)], idxb)
        pltpu.sync_copy(w_hbm.at[pl.ds(base, _EPT_EDGE)], wb)

        @pl.loop(0, _EPT_EDGE // _L)
        def _(i):
            sl = pl.ds(i * _L, _L)
            plsc.addupdate_scatter(hist, [idxb[sl]], wb[sl])

        pltpu.sync_copy(hist, out_hbm.at[wid])

    return k(dst, w)


_SLOTS = 5


def _make_sc_agg(col_split):
    ept = _EPT_COL if col_split else _EPT_EDGE
    n_ch = ept // _K
    n_blk = n_ch // _SLOTS

    @functools.partial(
        pl.kernel,
        out_type=jax.ShapeDtypeStruct((2 * _NP, 128), jnp.float32),
        mesh=_vmesh(),
        compiler_params=_sc_compiler_params(),
        scratch_types=[
            pltpu.VMEM_SHARED((_NP, 128), jnp.float32),
            pltpu.VMEM((_SLOTS, _K), jnp.int32),
            pltpu.VMEM((_SLOTS, _K), jnp.int32),
            pltpu.VMEM((_SLOTS, _K), jnp.float32),
            pltpu.VMEM((_SLOTS * _K, 128), jnp.float32),
            pltpu.VMEM((_ZR, 128), jnp.float32),
            pltpu.SemaphoreType.DMA((_SLOTS,)),
            pltpu.SemaphoreType.DMA((_SLOTS,)),
            pltpu.SemaphoreType.DMA((_SLOTS,)),
        ],
    )
    def k(h_hbm, src_hbm, dst_hbm, w_hbm, out_hbm,
          acc, sidx, didx, wb, rows, zbuf, si, sg, ss):
        c = lax.axis_index("c")
        s = lax.axis_index("s")
        zeros = jnp.zeros((_L,), jnp.float32)

        if col_split:
            ebase = s * _EPT_COL
            coff = c * _NP
        else:
            ebase = (c * _NS + s) * _EPT_EDGE
            coff = None

        def issue_idx(ci, sl):
            o = ebase + ci * _K
            pltpu.async_copy(src_hbm.at[pl.ds(o, _K)], sidx.at[sl], si.at[sl])
            pltpu.async_copy(dst_hbm.at[pl.ds(o, _K)], didx.at[sl], si.at[sl])
            pltpu.async_copy(w_hbm.at[pl.ds(o, _K)], wb.at[sl], si.at[sl])

        def start_gather(ci, sl):
            pltpu.make_async_copy(src_hbm.at[pl.ds(0, _K)], sidx.at[sl], si.at[sl]).wait()
            pltpu.make_async_copy(dst_hbm.at[pl.ds(0, _K)], didx.at[sl], si.at[sl]).wait()
            pltpu.make_async_copy(w_hbm.at[pl.ds(0, _K)], wb.at[sl], si.at[sl]).wait()
            if col_split:
                for j in range(_K // _L):
                    sv = sidx[sl, pl.ds(j * _L, _L)]
                    sidx[sl, pl.ds(j * _L, _L)] = sv + coff
                it = lax.broadcasted_iota(jnp.int32, (_L,), 0)
                cv = jnp.where(it >= (3 * _L - _K), coff, 0)
                sv = sidx[sl, pl.ds(_K - _L, _L)]
                sidx[sl, pl.ds(_K - _L, _L)] = sv + cv
            pltpu.async_copy(h_hbm.at[sidx.at[sl]],
                             rows.at[pl.ds(sl * _K, _K)], sg.at[sl])

        def finish(ci, sl):
            pltpu.make_async_copy(h_hbm.at[sidx.at[sl]],
                                  rows.at[pl.ds(sl * _K, _K)], sg.at[sl]).wait()

            @pl.loop(0, _K)
            def _(i):
                wv = plsc.load_gather(wb.at[sl], [jnp.full((_L,), i, jnp.int32)])
                r = sl * _K + i
                for j in range(128 // _L):
                    rows[r, pl.ds(j * _L, _L)] = rows[r, pl.ds(j * _L, _L)] * wv

            pltpu.async_copy(rows.at[pl.ds(sl * _K, _K)],
                             acc.at[didx.at[sl]], ss.at[sl], add=True)

        def retire(sl):
            pltpu.make_async_copy(rows.at[pl.ds(sl * _K, _K)],
                                  acc.at[didx.at[sl]], ss.at[sl]).wait()

        issue_idx(0, 0)
        issue_idx(1, 1)

        @pl.loop(0, _ZR)
        def _(r):
            for j in range(128 // _L):
                zbuf[r, pl.ds(j * _L, _L)] = zeros

        rbase = s * _RPT
        for t in range(_RPT // _ZR):
            pltpu.sync_copy(zbuf, acc.at[pl.ds(rbase + t * _ZR, _ZR)])
        plsc.subcore_barrier()

        start_gather(0, 0)
        for cp in range(_SLOTS):
            if cp >= 3:
                retire((cp + 2) % _SLOTS)
            issue_idx(cp + 2, (cp + 2) % _SLOTS)
            start_gather(cp + 1, (cp + 1) % _SLOTS)
            finish(cp, cp)

        @pl.loop(1, n_blk - 1)
        def _(blk):
            cb = blk * _SLOTS
            for r in range(_SLOTS):
                retire((r + 2) % _SLOTS)
                issue_idx(cb + r + 2, (r + 2) % _SLOTS)
                start_gather(cb + r + 1, (r + 1) % _SLOTS)
                finish(cb + r, r)

        for cp in range(n_ch - _SLOTS, n_ch):
            r = cp % _SLOTS
            retire((r + 2) % _SLOTS)
            if cp + 2 < n_ch:
                issue_idx(cp + 2, (r + 2) % _SLOTS)
            if cp + 1 < n_ch:
                start_gather(cp + 1, (r + 1) % _SLOTS)
            finish(cp, r)
        for sl in (2, 3, 4):
            retire(sl)

        plsc.subcore_barrier()
        coff_out = c * _NP
        for t in range(_RPT // _ZR):
            rr = rbase + t * _ZR
            pltpu.sync_copy(acc.at[pl.ds(rr, _ZR)],
                            out_hbm.at[pl.ds(coff_out + rr, _ZR)])

    return k


def _sc_agg_cols(h_flat, src, dst, w):
    return _make_sc_agg(True)(h_flat, src, dst, w)


def _sc_agg_edges(h3, src, dst, w):
    return _make_sc_agg(False)(h3, src, dst, w)


def _tc_dinv(degp):
    def body(deg_ref, o_ref):
        deg = jnp.sum(deg_ref[...], axis=0) + 1.0
        o_ref[...] = jnp.where(deg > 0.0,
                               lax.rsqrt(jnp.maximum(deg, 1e-12)), 0.0)

    return pl.pallas_call(
        body,
        out_shape=jax.ShapeDtypeStruct((128, 80), jnp.float32),
    )(degp)


def _tc_matmul(x, W):
    M, K = x.shape
    Nw = W.shape[1]

    def body(x_ref, w_ref, o_ref):
        o_ref[...] = jnp.dot(x_ref[...], w_ref[...],
                             preferred_element_type=jnp.float32)

    return pl.pallas_call(
        body,
        grid=(M // _R,),
        in_specs=[
            pl.BlockSpec((_R, K), lambda i: (i, 0)),
            pl.BlockSpec((K, Nw), lambda i: (0, 0)),
        ],
        out_specs=pl.BlockSpec((_R, Nw), lambda i: (i, 0)),
        out_shape=jax.ShapeDtypeStruct((M, Nw), jnp.float32),
    )(x, W)


def _tc_scale_split(g, dinv):
    def body(g_ref, dinv_ref, o_ref):
        h = g_ref[...] * dinv_ref[...]
        o_ref[0] = h[:, :128]
        o_ref[1] = h[:, 128:]

    return pl.pallas_call(
        body,
        grid=(_NP // _R,),
        in_specs=[
            pl.BlockSpec((_R, 256), lambda i: (i, 0)),
            pl.BlockSpec((_R, 1), lambda i: (i, 0)),
        ],
        out_specs=pl.BlockSpec((2, _R, 128), lambda i: (0, i, 0)),
        out_shape=jax.ShapeDtypeStruct((2, _NP, 128), jnp.float32),
    )(g, dinv)


def _tc_layer(acc, hs, dinv, b, W):
    Nw = W.shape[1]

    def body(acc_ref, hs_ref, dinv_ref, b_ref, w_ref, o_ref):
        dinv = dinv_ref[...]
        bb = b_ref[...]
        t0 = jnp.maximum(dinv * (acc_ref[0] + hs_ref[0]) + bb[:, :128], 0.0)
        t1 = jnp.maximum(dinv * (acc_ref[1] + hs_ref[1]) + bb[:, 128:], 0.0)
        t = jnp.concatenate([t0, t1], axis=1)
        g = jnp.dot(t, w_ref[...], preferred_element_type=jnp.float32)
        h = g * dinv
        if Nw == 256:
            o_ref[0] = h[:, :128]
            o_ref[1] = h[:, 128:]
        else:
            o_ref[...] = h

    if Nw == 256:
        out_spec = pl.BlockSpec((2, _R, 128), lambda i: (0, i, 0))
        out_shape = jax.ShapeDtypeStruct((2, _NP, 128), jnp.float32)
    else:
        out_spec = pl.BlockSpec((_R, Nw), lambda i: (i, 0))
        out_shape = jax.ShapeDtypeStruct((_NP, Nw), jnp.float32)

    return pl.pallas_call(
        body,
        grid=(_NP // _R,),
        in_specs=[
            pl.BlockSpec((2, _R, 128), lambda i: (0, i, 0)),
            pl.BlockSpec((2, _R, 128), lambda i: (0, i, 0)),
            pl.BlockSpec((_R, 1), lambda i: (i, 0)),
            pl.BlockSpec((1, 256), lambda i: (0, 0)),
            pl.BlockSpec(W.shape, lambda i: (0, 0)),
        ],
        out_specs=out_spec,
        out_shape=out_shape,
    )(acc, hs, dinv, b, W)


def _tc_final(accp, h3, dinv, b3):
    def body(acc_ref, h_ref, dinv_ref, b_ref, o_ref):
        sagg = acc_ref[0] + acc_ref[1] + h_ref[...]
        z = (dinv_ref[...] * sagg + b_ref[...])[:, :40]
        m = jnp.max(z, axis=1, keepdims=True)
        ze = z - m
        lse = jnp.log(jnp.sum(jnp.exp(ze), axis=1, keepdims=True))
        o_ref[...] = ze - lse

    return pl.pallas_call(
        body,
        grid=(_NP // _R,),
        in_specs=[
            pl.BlockSpec((2, _R, 128), lambda i: (0, i, 0)),
            pl.BlockSpec((_R, 128), lambda i: (i, 0)),
            pl.BlockSpec((_R, 1), lambda i: (i, 0)),
            pl.BlockSpec((1, 128), lambda i: (0, 0)),
        ],
        out_specs=pl.BlockSpec((_R, 40), lambda i: (i, 0)),
        out_shape=jax.ShapeDtypeStruct((_NP, 40), jnp.float32),
    )(accp, h3, dinv, b3)


def kernel(x, edge_index, edge_attr, W1, b1, W2, b2, W3, b3):
    src = edge_index[0].astype(jnp.int32)
    dst = edge_index[1].astype(jnp.int32)
    w = edge_attr.astype(jnp.float32)
    xp = jnp.pad(x, ((0, _NP - _N), (0, 0)))

    degp = _sc_degree(dst, w)
    g1 = _tc_matmul(xp, W1)
    dinv = _tc_dinv(degp.reshape(_NW, 128, 80)).reshape(_NP, 1)

    h1 = _tc_scale_split(g1, dinv)
    acc1 = _sc_agg_cols(h1.reshape(2 * _NP, 128), src, dst, w)
    h2 = _tc_layer(acc1.reshape(2, _NP, 128), h1, dinv,
                   b1.reshape(1, 256), W2)

    acc2 = _sc_agg_cols(h2.reshape(2 * _NP, 128), src, dst, w)
    W3p = jnp.pad(W3, ((0, 0), (0, 88)))
    h3 = _tc_layer(acc2.reshape(2, _NP, 128), h2, dinv,
                   b2.reshape(1, 256), W3p)

    acc3 = _sc_agg_edges(h3, src, dst, w)
    b3p = jnp.pad(b3, (0, 88)).reshape(1, 128)
    out = _tc_final(acc3.reshape(2, _NP, 128), h3, dinv, b3p)
    return out[:_N]


# speedup vs baseline: 1.3075x; 1.0021x over previous
import dataclasses
import functools

import jax
import jax.numpy as jnp
from jax import lax
from jax.experimental import pallas as pl
from jax.experimental.pallas import tpu as pltpu
from jax.experimental.pallas import tpu_sc as plsc

_N = 10000
_NP = 10240
_E = 320000
_NC, _NS, _L = 2, 16, 16
_NW = _NC * _NS

_K = 40
_EPT_COL = _E // _NS
_EPT_EDGE = _E // _NW
_RPT = _NP // _NS
_ZR = 64

_R = 2048


def _sc_compiler_params():
    cp = pltpu.CompilerParams()
    if "needs_layout_passes" in pltpu.CompilerParams.__dataclass_fields__:
        cp = dataclasses.replace(cp, needs_layout_passes=False)
    return cp


def _vmesh():
    return plsc.VectorSubcoreMesh(core_axis_name="c", subcore_axis_name="s")


def _sc_degree(dst, w):
    @functools.partial(
        pl.kernel,
        out_type=jax.ShapeDtypeStruct((_NW, _NP), jnp.float32),
        mesh=_vmesh(),
        compiler_params=_sc_compiler_params(),
        scratch_types=[
            pltpu.VMEM((_NP,), jnp.float32),
            pltpu.VMEM((_EPT_EDGE,), jnp.int32),
            pltpu.VMEM((_EPT_EDGE,), jnp.float32),
        ],
    )
    def k(dst_hbm, w_hbm, out_hbm, hist, idxb, wb):
        c = lax.axis_index("c")
        s = lax.axis_index("s")
        wid = c * _NS + s
        zeros = jnp.zeros((_L,), jnp.float32)

        @pl.loop(0, _NP // _L)
        def _(i):
            hist[pl.ds(i * _L, _L)] = zeros

        base = wid * _EPT_EDGE
        pltpu.sync_copy(dst_hbm.at[pl.ds(base, _EPT_EDGE)], idxb)
        pltpu.sync_copy(w_hbm.at[pl.ds(base, _EPT_EDGE)], wb)

        @pl.loop(0, _EPT_EDGE // _L)
        def _(i):
            sl = pl.ds(i * _L, _L)
            plsc.addupdate_scatter(hist, [idxb[sl]], wb[sl])

        pltpu.sync_copy(hist, out_hbm.at[wid])

    return k(dst, w)


_SLOTS = 5


def _make_sc_agg(col_split):
    ept = _EPT_COL if col_split else _EPT_EDGE
    n_ch = ept // _K
    n_blk = n_ch // _SLOTS

    @functools.partial(
        pl.kernel,
        out_type=jax.ShapeDtypeStruct((2 * _NP, 128), jnp.float32),
        mesh=_vmesh(),
        compiler_params=_sc_compiler_params(),
        scratch_types=[
            pltpu.VMEM_SHARED((_NP, 128), jnp.float32),
            pltpu.VMEM((_SLOTS, _K), jnp.int32),
            pltpu.VMEM((_SLOTS, _K), jnp.int32),
            pltpu.VMEM((_SLOTS, _K), jnp.float32),
            pltpu.VMEM((_SLOTS * _K, 128), jnp.float32),
            pltpu.VMEM((_ZR, 128), jnp.float32),
            pltpu.SemaphoreType.DMA((_SLOTS,)),
            pltpu.SemaphoreType.DMA((_SLOTS,)),
            pltpu.SemaphoreType.DMA((_SLOTS,)),
        ],
    )
    def k(h_hbm, src_hbm, dst_hbm, w_hbm, out_hbm,
          acc, sidx, didx, wb, rows, zbuf, si, sg, ss):
        c = lax.axis_index("c")
        s = lax.axis_index("s")
        zeros = jnp.zeros((_L,), jnp.float32)

        if col_split:
            ebase = s * _EPT_COL
            coff = c * _NP
        else:
            ebase = (c * _NS + s) * _EPT_EDGE
            coff = None

        def issue_idx(ci, sl):
            o = ebase + ci * _K
            pltpu.async_copy(src_hbm.at[pl.ds(o, _K)], sidx.at[sl], si.at[sl])
            pltpu.async_copy(dst_hbm.at[pl.ds(o, _K)], didx.at[sl], si.at[sl])
            pltpu.async_copy(w_hbm.at[pl.ds(o, _K)], wb.at[sl], si.at[sl])

        def start_gather(ci, sl):
            pltpu.make_async_copy(src_hbm.at[pl.ds(0, _K)], sidx.at[sl], si.at[sl]).wait()
            pltpu.make_async_copy(dst_hbm.at[pl.ds(0, _K)], didx.at[sl], si.at[sl]).wait()
            pltpu.make_async_copy(w_hbm.at[pl.ds(0, _K)], wb.at[sl], si.at[sl]).wait()
            if col_split:
                for j in range(_K // _L):
                    sv = sidx[sl, pl.ds(j * _L, _L)]
                    sidx[sl, pl.ds(j * _L, _L)] = sv + coff
                it = lax.broadcasted_iota(jnp.int32, (_L,), 0)
                cv = jnp.where(it >= (3 * _L - _K), coff, 0)
                sv = sidx[sl, pl.ds(_K - _L, _L)]
                sidx[sl, pl.ds(_K - _L, _L)] = sv + cv
            pltpu.async_copy(h_hbm.at[sidx.at[sl]],
                             rows.at[pl.ds(sl * _K, _K)], sg.at[sl])

        def finish(ci, sl):
            pltpu.make_async_copy(h_hbm.at[sidx.at[sl]],
                                  rows.at[pl.ds(sl * _K, _K)], sg.at[sl]).wait()

            @pl.loop(0, _K)
            def _(i):
                wv = plsc.load_gather(wb.at[sl], [jnp.full((_L,), i, jnp.int32)])
                r = sl * _K + i
                for j in range(128 // _L):
                    rows[r, pl.ds(j * _L, _L)] = rows[r, pl.ds(j * _L, _L)] * wv

            pltpu.async_copy(rows.at[pl.ds(sl * _K, _K)],
                             acc.at[didx.at[sl]], ss.at[sl], add=True)

        def retire(sl):
            pltpu.make_async_copy(rows.at[pl.ds(sl * _K, _K)],
                                  acc.at[didx.at[sl]], ss.at[sl]).wait()

        issue_idx(0, 0)
        issue_idx(1, 1)

        @pl.loop(0, _ZR)
        def _(r):
            for j in range(128 // _L):
                zbuf[r, pl.ds(j * _L, _L)] = zeros

        rbase = s * _RPT
        for t in range(_RPT // _ZR):
            pltpu.async_copy(zbuf, acc.at[pl.ds(rbase + t * _ZR, _ZR)],
                             ss.at[0])
        for t in range(_RPT // _ZR):
            pltpu.make_async_copy(zbuf, acc.at[pl.ds(rbase, _ZR)],
                                  ss.at[0]).wait()
        plsc.subcore_barrier()

        start_gather(0, 0)
        for cp in range(_SLOTS):
            if cp >= 3:
                retire((cp + 2) % _SLOTS)
            issue_idx(cp + 2, (cp + 2) % _SLOTS)
            start_gather(cp + 1, (cp + 1) % _SLOTS)
            finish(cp, cp)

        @pl.loop(1, n_blk - 1)
        def _(blk):
            cb = blk * _SLOTS
            for r in range(_SLOTS):
                retire((r + 2) % _SLOTS)
                issue_idx(cb + r + 2, (r + 2) % _SLOTS)
                start_gather(cb + r + 1, (r + 1) % _SLOTS)
                finish(cb + r, r)

        for cp in range(n_ch - _SLOTS, n_ch):
            r = cp % _SLOTS
            retire((r + 2) % _SLOTS)
            if cp + 2 < n_ch:
                issue_idx(cp + 2, (r + 2) % _SLOTS)
            if cp + 1 < n_ch:
                start_gather(cp + 1, (r + 1) % _SLOTS)
            finish(cp, r)
        for sl in (2, 3, 4):
            retire(sl)

        plsc.subcore_barrier()
        coff_out = c * _NP
        for t in range(_RPT // _ZR):
            rr = rbase + t * _ZR
            pltpu.async_copy(acc.at[pl.ds(rr, _ZR)],
                             out_hbm.at[pl.ds(coff_out + rr, _ZR)], ss.at[0])
        for t in range(_RPT // _ZR):
            pltpu.make_async_copy(acc.at[pl.ds(rbase, _ZR)],
                                  out_hbm.at[pl.ds(coff_out + rbase, _ZR)],
                                  ss.at[0]).wait()

    return k


def _sc_agg_cols(h_flat, src, dst, w):
    return _make_sc_agg(True)(h_flat, src, dst, w)


def _sc_agg_edges(h3, src, dst, w):
    return _make_sc_agg(False)(h3, src, dst, w)


def _tc_dinv(degp):
    def body(deg_ref, o_ref):
        deg = jnp.sum(deg_ref[...], axis=0) + 1.0
        o_ref[...] = jnp.where(deg > 0.0,
                               lax.rsqrt(jnp.maximum(deg, 1e-12)), 0.0)

    return pl.pallas_call(
        body,
        out_shape=jax.ShapeDtypeStruct((128, 80), jnp.float32),
    )(degp)


def _tc_matmul(x, W):
    M, K = x.shape
    Nw = W.shape[1]

    def body(x_ref, w_ref, o_ref):
        o_ref[...] = jnp.dot(x_ref[...], w_ref[...],
                             preferred_element_type=jnp.float32)

    return pl.pallas_call(
        body,
        grid=(M // _R,),
        in_specs=[
            pl.BlockSpec((_R, K), lambda i: (i, 0)),
            pl.BlockSpec((K, Nw), lambda i: (0, 0)),
        ],
        out_specs=pl.BlockSpec((_R, Nw), lambda i: (i, 0)),
        out_shape=jax.ShapeDtypeStruct((M, Nw), jnp.float32),
    )(x, W)


def _tc_scale_split(g, dinv):
    def body(g_ref, dinv_ref, o_ref):
        h = g_ref[...] * dinv_ref[...]
        o_ref[0] = h[:, :128]
        o_ref[1] = h[:, 128:]

    return pl.pallas_call(
        body,
        grid=(_NP // _R,),
        in_specs=[
            pl.BlockSpec((_R, 256), lambda i: (i, 0)),
            pl.BlockSpec((_R, 1), lambda i: (i, 0)),
        ],
        out_specs=pl.BlockSpec((2, _R, 128), lambda i: (0, i, 0)),
        out_shape=jax.ShapeDtypeStruct((2, _NP, 128), jnp.float32),
    )(g, dinv)


def _tc_layer(acc, hs, dinv, b, W):
    Nw = W.shape[1]

    def body(acc_ref, hs_ref, dinv_ref, b_ref, w_ref, o_ref):
        dinv = dinv_ref[...]
        bb = b_ref[...]
        t0 = jnp.maximum(dinv * (acc_ref[0] + hs_ref[0]) + bb[:, :128], 0.0)
        t1 = jnp.maximum(dinv * (acc_ref[1] + hs_ref[1]) + bb[:, 128:], 0.0)
        t = jnp.concatenate([t0, t1], axis=1)
        g = jnp.dot(t, w_ref[...], preferred_element_type=jnp.float32)
        h = g * dinv
        if Nw == 256:
            o_ref[0] = h[:, :128]
            o_ref[1] = h[:, 128:]
        else:
            o_ref[...] = h

    if Nw == 256:
        out_spec = pl.BlockSpec((2, _R, 128), lambda i: (0, i, 0))
        out_shape = jax.ShapeDtypeStruct((2, _NP, 128), jnp.float32)
    else:
        out_spec = pl.BlockSpec((_R, Nw), lambda i: (i, 0))
        out_shape = jax.ShapeDtypeStruct((_NP, Nw), jnp.float32)

    return pl.pallas_call(
        body,
        grid=(_NP // _R,),
        in_specs=[
            pl.BlockSpec((2, _R, 128), lambda i: (0, i, 0)),
            pl.BlockSpec((2, _R, 128), lambda i: (0, i, 0)),
            pl.BlockSpec((_R, 1), lambda i: (i, 0)),
            pl.BlockSpec((1, 256), lambda i: (0, 0)),
            pl.BlockSpec(W.shape, lambda i: (0, 0)),
        ],
        out_specs=out_spec,
        out_shape=out_shape,
    )(acc, hs, dinv, b, W)


def _tc_final(accp, h3, dinv, b3):
    def body(acc_ref, h_ref, dinv_ref, b_ref, o_ref):
        sagg = acc_ref[0] + acc_ref[1] + h_ref[...]
        z = (dinv_ref[...] * sagg + b_ref[...])[:, :40]
        m = jnp.max(z, axis=1, keepdims=True)
        ze = z - m
        lse = jnp.log(jnp.sum(jnp.exp(ze), axis=1, keepdims=True))
        o_ref[...] = ze - lse

    return pl.pallas_call(
        body,
        grid=(_NP // _R,),
        in_specs=[
            pl.BlockSpec((2, _R, 128), lambda i: (0, i, 0)),
            pl.BlockSpec((_R, 128), lambda i: (i, 0)),
            pl.BlockSpec((_R, 1), lambda i: (i, 0)),
            pl.BlockSpec((1, 128), lambda i: (0, 0)),
        ],
        out_specs=pl.BlockSpec((_R, 40), lambda i: (i, 0)),
        out_shape=jax.ShapeDtypeStruct((_NP, 40), jnp.float32),
    )(accp, h3, dinv, b3)


def kernel(x, edge_index, edge_attr, W1, b1, W2, b2, W3, b3):
    src = edge_index[0].astype(jnp.int32)
    dst = edge_index[1].astype(jnp.int32)
    w = edge_attr.astype(jnp.float32)
    xp = jnp.pad(x, ((0, _NP - _N), (0, 0)))

    degp = _sc_degree(dst, w)
    g1 = _tc_matmul(xp, W1)
    dinv = _tc_dinv(degp.reshape(_NW, 128, 80)).reshape(_NP, 1)

    h1 = _tc_scale_split(g1, dinv)
    acc1 = _sc_agg_cols(h1.reshape(2 * _NP, 128), src, dst, w)
    h2 = _tc_layer(acc1.reshape(2, _NP, 128), h1, dinv,
                   b1.reshape(1, 256), W2)

    acc2 = _sc_agg_cols(h2.reshape(2 * _NP, 128), src, dst, w)
    W3p = jnp.pad(W3, ((0, 0), (0, 88)))
    h3 = _tc_layer(acc2.reshape(2, _NP, 128), h2, dinv,
                   b2.reshape(1, 256), W3p)

    acc3 = _sc_agg_edges(h3, src, dst, w)
    b3p = jnp.pad(b3, (0, 88)).reshape(1, 128)
    out = _tc_final(acc3.reshape(2, _NP, 128), h3, dinv, b3p)
    return out[:_N]


# speedup vs baseline: 1.4115x; 1.0795x over previous
import dataclasses
import functools

import jax
import jax.numpy as jnp
from jax import lax
from jax.experimental import pallas as pl
from jax.experimental.pallas import tpu as pltpu
from jax.experimental.pallas import tpu_sc as plsc

_N = 10000
_NP = 10240
_E = 320000
_NC, _NS, _L = 2, 16, 16
_NW = _NC * _NS

_K = 40
_EPT_COL = _E // _NS
_EPT_EDGE = _E // _NW
_RPT = _NP // _NS
_ZR = 64

_R = 2048


def _sc_compiler_params():
    cp = pltpu.CompilerParams()
    if "needs_layout_passes" in pltpu.CompilerParams.__dataclass_fields__:
        cp = dataclasses.replace(cp, needs_layout_passes=False)
    return cp


def _vmesh():
    return plsc.VectorSubcoreMesh(core_axis_name="c", subcore_axis_name="s")


def _sc_degree(dst, w):
    @functools.partial(
        pl.kernel,
        out_type=jax.ShapeDtypeStruct((_NW, _NP), jnp.float32),
        mesh=_vmesh(),
        compiler_params=_sc_compiler_params(),
        scratch_types=[
            pltpu.VMEM((_NP,), jnp.float32),
            pltpu.VMEM((_EPT_EDGE,), jnp.int32),
            pltpu.VMEM((_EPT_EDGE,), jnp.float32),
        ],
    )
    def k(dst_hbm, w_hbm, out_hbm, hist, idxb, wb):
        c = lax.axis_index("c")
        s = lax.axis_index("s")
        wid = c * _NS + s
        zeros = jnp.zeros((_L,), jnp.float32)

        @pl.loop(0, _NP // _L)
        def _(i):
            hist[pl.ds(i * _L, _L)] = zeros

        base = wid * _EPT_EDGE
        pltpu.sync_copy(dst_hbm.at[pl.ds(base, _EPT_EDGE)], idxb)
        pltpu.sync_copy(w_hbm.at[pl.ds(base, _EPT_EDGE)], wb)

        @pl.loop(0, _EPT_EDGE // _L)
        def _(i):
            sl = pl.ds(i * _L, _L)
            plsc.addupdate_scatter(hist, [idxb[sl]], wb[sl])

        pltpu.sync_copy(hist, out_hbm.at[wid])

    return k(dst, w)


_SLOTS = 5


def _make_sc_agg(col_split):
    ept = _EPT_COL if col_split else _EPT_EDGE
    n_ch = ept // _K
    n_blk = n_ch // _SLOTS

    @functools.partial(
        pl.kernel,
        out_type=jax.ShapeDtypeStruct((2 * _NP, 128), jnp.float32),
        mesh=_vmesh(),
        compiler_params=_sc_compiler_params(),
        scratch_types=[
            pltpu.VMEM_SHARED((_NP, 128), jnp.float32),
            pltpu.VMEM((_SLOTS, _K), jnp.int32),
            pltpu.VMEM((_SLOTS, _K), jnp.int32),
            pltpu.VMEM((_SLOTS, _K), jnp.float32),
            pltpu.VMEM((_SLOTS * _K, 128), jnp.float32),
            pltpu.VMEM((_ZR, 128), jnp.float32),
            pltpu.SemaphoreType.DMA((_SLOTS,)),
            pltpu.SemaphoreType.DMA((_SLOTS,)),
            pltpu.SemaphoreType.DMA((_SLOTS,)),
        ],
    )
    def k(h_hbm, src_hbm, dst_hbm, w_hbm, out_hbm,
          acc, sidx, didx, wb, rows, zbuf, si, sg, ss):
        c = lax.axis_index("c")
        s = lax.axis_index("s")
        zeros = jnp.zeros((_L,), jnp.float32)

        if col_split:
            ebase = s * _EPT_COL
            coff = c * _NP
        else:
            ebase = (c * _NS + s) * _EPT_EDGE
            coff = None

        def issue_idx(ci, sl):
            o = ebase + ci * _K
            pltpu.async_copy(src_hbm.at[pl.ds(o, _K)], sidx.at[sl], si.at[sl])
            pltpu.async_copy(dst_hbm.at[pl.ds(o, _K)], didx.at[sl], si.at[sl])
            pltpu.async_copy(w_hbm.at[pl.ds(o, _K)], wb.at[sl], si.at[sl])

        def start_gather(ci, sl):
            pltpu.make_async_copy(src_hbm.at[pl.ds(0, _K)], sidx.at[sl], si.at[sl]).wait()
            pltpu.make_async_copy(dst_hbm.at[pl.ds(0, _K)], didx.at[sl], si.at[sl]).wait()
            pltpu.make_async_copy(w_hbm.at[pl.ds(0, _K)], wb.at[sl], si.at[sl]).wait()
            if col_split:
                for j in range(_K // _L):
                    sv = sidx[sl, pl.ds(j * _L, _L)]
                    sidx[sl, pl.ds(j * _L, _L)] = sv + coff
                it = lax.broadcasted_iota(jnp.int32, (_L,), 0)
                cv = jnp.where(it >= (3 * _L - _K), coff, 0)
                sv = sidx[sl, pl.ds(_K - _L, _L)]
                sidx[sl, pl.ds(_K - _L, _L)] = sv + cv
            pltpu.async_copy(h_hbm.at[sidx.at[sl]],
                             rows.at[pl.ds(sl * _K, _K)], sg.at[sl])

        def finish(ci, sl):
            pltpu.make_async_copy(h_hbm.at[sidx.at[sl]],
                                  rows.at[pl.ds(sl * _K, _K)], sg.at[sl]).wait()

            @pl.loop(0, _K // 2)
            def _(i):
                i0 = 2 * i
                wv0 = plsc.load_gather(wb.at[sl], [jnp.full((_L,), i0, jnp.int32)])
                wv1 = plsc.load_gather(wb.at[sl], [jnp.full((_L,), i0 + 1, jnp.int32)])
                r = sl * _K + i0
                for j in range(128 // _L):
                    rows[r, pl.ds(j * _L, _L)] = rows[r, pl.ds(j * _L, _L)] * wv0
                    rows[r + 1, pl.ds(j * _L, _L)] = rows[r + 1, pl.ds(j * _L, _L)] * wv1

            pltpu.async_copy(rows.at[pl.ds(sl * _K, _K)],
                             acc.at[didx.at[sl]], ss.at[sl], add=True)

        def retire(sl):
            pltpu.make_async_copy(rows.at[pl.ds(sl * _K, _K)],
                                  acc.at[didx.at[sl]], ss.at[sl]).wait()

        issue_idx(0, 0)
        issue_idx(1, 1)

        @pl.loop(0, _ZR)
        def _(r):
            for j in range(128 // _L):
                zbuf[r, pl.ds(j * _L, _L)] = zeros

        rbase = s * _RPT
        for t in range(_RPT // _ZR):
            pltpu.async_copy(zbuf, acc.at[pl.ds(rbase + t * _ZR, _ZR)],
                             ss.at[0])
        for t in range(_RPT // _ZR):
            pltpu.make_async_copy(zbuf, acc.at[pl.ds(rbase, _ZR)],
                                  ss.at[0]).wait()
        plsc.subcore_barrier()

        start_gather(0, 0)
        for cp in range(_SLOTS):
            if cp >= 3:
                retire((cp + 2) % _SLOTS)
            issue_idx(cp + 2, (cp + 2) % _SLOTS)
            start_gather(cp + 1, (cp + 1) % _SLOTS)
            finish(cp, cp)

        @pl.loop(1, n_blk - 1)
        def _(blk):
            cb = blk * _SLOTS
            for r in range(_SLOTS):
                retire((r + 2) % _SLOTS)
                issue_idx(cb + r + 2, (r + 2) % _SLOTS)
                start_gather(cb + r + 1, (r + 1) % _SLOTS)
                finish(cb + r, r)

        for cp in range(n_ch - _SLOTS, n_ch):
            r = cp % _SLOTS
            retire((r + 2) % _SLOTS)
            if cp + 2 < n_ch:
                issue_idx(cp + 2, (r + 2) % _SLOTS)
            if cp + 1 < n_ch:
                start_gather(cp + 1, (r + 1) % _SLOTS)
            finish(cp, r)
        for sl in (2, 3, 4):
            retire(sl)

        plsc.subcore_barrier()
        coff_out = c * _NP
        for t in range(_RPT // _ZR):
            rr = rbase + t * _ZR
            pltpu.async_copy(acc.at[pl.ds(rr, _ZR)],
                             out_hbm.at[pl.ds(coff_out + rr, _ZR)], ss.at[0])
        for t in range(_RPT // _ZR):
            pltpu.make_async_copy(acc.at[pl.ds(rbase, _ZR)],
                                  out_hbm.at[pl.ds(coff_out + rbase, _ZR)],
                                  ss.at[0]).wait()

    return k


def _sc_agg_cols(h_flat, src, dst, w):
    return _make_sc_agg(True)(h_flat, src, dst, w)


def _sc_agg_edges(h3, src, dst, w):
    return _make_sc_agg(False)(h3, src, dst, w)


def _tc_dinv(degp):
    def body(deg_ref, o_ref):
        deg = jnp.sum(deg_ref[...], axis=0) + 1.0
        o_ref[...] = jnp.where(deg > 0.0,
                               lax.rsqrt(jnp.maximum(deg, 1e-12)), 0.0)

    return pl.pallas_call(
        body,
        out_shape=jax.ShapeDtypeStruct((128, 80), jnp.float32),
    )(degp)


def _tc_matmul(x, W):
    M, K = x.shape
    Nw = W.shape[1]

    def body(x_ref, w_ref, o_ref):
        o_ref[...] = jnp.dot(x_ref[...], w_ref[...],
                             preferred_element_type=jnp.float32)

    return pl.pallas_call(
        body,
        grid=(M // _R,),
        in_specs=[
            pl.BlockSpec((_R, K), lambda i: (i, 0)),
            pl.BlockSpec((K, Nw), lambda i: (0, 0)),
        ],
        out_specs=pl.BlockSpec((_R, Nw), lambda i: (i, 0)),
        out_shape=jax.ShapeDtypeStruct((M, Nw), jnp.float32),
    )(x, W)


def _tc_scale_split(g, dinv):
    def body(g_ref, dinv_ref, o_ref):
        h = g_ref[...] * dinv_ref[...]
        o_ref[0] = h[:, :128]
        o_ref[1] = h[:, 128:]

    return pl.pallas_call(
        body,
        grid=(_NP // _R,),
        in_specs=[
            pl.BlockSpec((_R, 256), lambda i: (i, 0)),
            pl.BlockSpec((_R, 1), lambda i: (i, 0)),
        ],
        out_specs=pl.BlockSpec((2, _R, 128), lambda i: (0, i, 0)),
        out_shape=jax.ShapeDtypeStruct((2, _NP, 128), jnp.float32),
    )(g, dinv)


def _tc_layer(acc, hs, dinv, b, W):
    Nw = W.shape[1]

    def body(acc_ref, hs_ref, dinv_ref, b_ref, w_ref, o_ref):
        dinv = dinv_ref[...]
        bb = b_ref[...]
        t0 = jnp.maximum(dinv * (acc_ref[0] + hs_ref[0]) + bb[:, :128], 0.0)
        t1 = jnp.maximum(dinv * (acc_ref[1] + hs_ref[1]) + bb[:, 128:], 0.0)
        t = jnp.concatenate([t0, t1], axis=1)
        g = jnp.dot(t, w_ref[...], preferred_element_type=jnp.float32)
        h = g * dinv
        if Nw == 256:
            o_ref[0] = h[:, :128]
            o_ref[1] = h[:, 128:]
        else:
            o_ref[...] = h

    if Nw == 256:
        out_spec = pl.BlockSpec((2, _R, 128), lambda i: (0, i, 0))
        out_shape = jax.ShapeDtypeStruct((2, _NP, 128), jnp.float32)
    else:
        out_spec = pl.BlockSpec((_R, Nw), lambda i: (i, 0))
        out_shape = jax.ShapeDtypeStruct((_NP, Nw), jnp.float32)

    return pl.pallas_call(
        body,
        grid=(_NP // _R,),
        in_specs=[
            pl.BlockSpec((2, _R, 128), lambda i: (0, i, 0)),
            pl.BlockSpec((2, _R, 128), lambda i: (0, i, 0)),
            pl.BlockSpec((_R, 1), lambda i: (i, 0)),
            pl.BlockSpec((1, 256), lambda i: (0, 0)),
            pl.BlockSpec(W.shape, lambda i: (0, 0)),
        ],
        out_specs=out_spec,
        out_shape=out_shape,
    )(acc, hs, dinv, b, W)


def _tc_final(accp, h3, dinv, b3):
    def body(acc_ref, h_ref, dinv_ref, b_ref, o_ref):
        sagg = acc_ref[0] + acc_ref[1] + h_ref[...]
        z = (dinv_ref[...] * sagg + b_ref[...])[:, :40]
        m = jnp.max(z, axis=1, keepdims=True)
        ze = z - m
        lse = jnp.log(jnp.sum(jnp.exp(ze), axis=1, keepdims=True))
        o_ref[...] = ze - lse

    return pl.pallas_call(
        body,
        grid=(_NP // _R,),
        in_specs=[
            pl.BlockSpec((2, _R, 128), lambda i: (0, i, 0)),
            pl.BlockSpec((_R, 128), lambda i: (i, 0)),
            pl.BlockSpec((_R, 1), lambda i: (i, 0)),
            pl.BlockSpec((1, 128), lambda i: (0, 0)),
        ],
        out_specs=pl.BlockSpec((_R, 40), lambda i: (i, 0)),
        out_shape=jax.ShapeDtypeStruct((_NP, 40), jnp.float32),
    )(accp, h3, dinv, b3)


def kernel(x, edge_index, edge_attr, W1, b1, W2, b2, W3, b3):
    src = edge_index[0].astype(jnp.int32)
    dst = edge_index[1].astype(jnp.int32)
    w = edge_attr.astype(jnp.float32)
    xp = jnp.pad(x, ((0, _NP - _N), (0, 0)))

    degp = _sc_degree(dst, w)
    g1 = _tc_matmul(xp, W1)
    dinv = _tc_dinv(degp.reshape(_NW, 128, 80)).reshape(_NP, 1)

    h1 = _tc_scale_split(g1, dinv)
    acc1 = _sc_agg_cols(h1.reshape(2 * _NP, 128), src, dst, w)
    h2 = _tc_layer(acc1.reshape(2, _NP, 128), h1, dinv,
                   b1.reshape(1, 256), W2)

    acc2 = _sc_agg_cols(h2.reshape(2 * _NP, 128), src, dst, w)
    W3p = jnp.pad(W3, ((0, 0), (0, 88)))
    h3 = _tc_layer(acc2.reshape(2, _NP, 128), h2, dinv,
                   b2.reshape(1, 256), W3p)

    acc3 = _sc_agg_edges(h3, src, dst, w)
    b3p = jnp.pad(b3, (0, 88)).reshape(1, 128)
    out = _tc_final(acc3.reshape(2, _NP, 128), h3, dinv, b3p)
    return out[:_N]


# speedup vs baseline: 1.4418x; 1.0215x over previous
import dataclasses
import functools

import jax
import jax.numpy as jnp
from jax import lax
from jax.experimental import pallas as pl
from jax.experimental.pallas import tpu as pltpu
from jax.experimental.pallas import tpu_sc as plsc

_N = 10000
_NP = 10240
_E = 320000
_NC, _NS, _L = 2, 16, 16
_NW = _NC * _NS

_K = 40
_EPT_COL = _E // _NS
_EPT_EDGE = _E // _NW
_RPT = _NP // _NS
_ZR = 64

_R = 2048


def _sc_compiler_params():
    cp = pltpu.CompilerParams()
    if "needs_layout_passes" in pltpu.CompilerParams.__dataclass_fields__:
        cp = dataclasses.replace(cp, needs_layout_passes=False)
    return cp


def _vmesh():
    return plsc.VectorSubcoreMesh(core_axis_name="c", subcore_axis_name="s")


def _sc_degree(dst, w):
    @functools.partial(
        pl.kernel,
        out_type=jax.ShapeDtypeStruct((_NW, _NP), jnp.float32),
        mesh=_vmesh(),
        compiler_params=_sc_compiler_params(),
        scratch_types=[
            pltpu.VMEM((_NP,), jnp.float32),
            pltpu.VMEM((_EPT_EDGE,), jnp.int32),
            pltpu.VMEM((_EPT_EDGE,), jnp.float32),
        ],
    )
    def k(dst_hbm, w_hbm, out_hbm, hist, idxb, wb):
        c = lax.axis_index("c")
        s = lax.axis_index("s")
        wid = c * _NS + s
        zeros = jnp.zeros((_L,), jnp.float32)

        @pl.loop(0, _NP // _L)
        def _(i):
            hist[pl.ds(i * _L, _L)] = zeros

        base = wid * _EPT_EDGE
        pltpu.sync_copy(dst_hbm.at[pl.ds(base, _EPT_EDGE)], idxb)
        pltpu.sync_copy(w_hbm.at[pl.ds(base, _EPT_EDGE)], wb)

        @pl.loop(0, _EPT_EDGE // _L)
        def _(i):
            sl = pl.ds(i * _L, _L)
            plsc.addupdate_scatter(hist, [idxb[sl]], wb[sl])

        pltpu.sync_copy(hist, out_hbm.at[wid])

    return k(dst, w)


_SLOTS = 5


def _make_sc_agg(col_split):
    ept = _EPT_COL if col_split else _EPT_EDGE
    n_ch = ept // _K
    n_blk = n_ch // _SLOTS

    @functools.partial(
        pl.kernel,
        out_type=jax.ShapeDtypeStruct((2 * _NP, 128), jnp.float32),
        mesh=_vmesh(),
        compiler_params=_sc_compiler_params(),
        scratch_types=[
            pltpu.VMEM_SHARED((_NP, 128), jnp.float32),
            pltpu.VMEM((_SLOTS, _K), jnp.int32),
            pltpu.VMEM((_SLOTS, _K), jnp.int32),
            pltpu.VMEM((_SLOTS, _K), jnp.float32),
            pltpu.VMEM((_SLOTS * _K, 128), jnp.float32),
            pltpu.VMEM((_ZR, 128), jnp.float32),
            pltpu.SemaphoreType.DMA((_SLOTS,)),
            pltpu.SemaphoreType.DMA((_SLOTS,)),
            pltpu.SemaphoreType.DMA((_SLOTS,)),
        ],
    )
    def k(h_hbm, src_hbm, dst_hbm, w_hbm, out_hbm,
          acc, sidx, didx, wb, rows, zbuf, si, sg, ss):
        c = lax.axis_index("c")
        s = lax.axis_index("s")
        zeros = jnp.zeros((_L,), jnp.float32)

        if col_split:
            ebase = s * _EPT_COL
            coff = c * _NP
        else:
            ebase = (c * _NS + s) * _EPT_EDGE
            coff = None

        def issue_idx(ci, sl):
            o = ebase + ci * _K
            pltpu.async_copy(src_hbm.at[pl.ds(o, _K)], sidx.at[sl], si.at[sl])
            pltpu.async_copy(dst_hbm.at[pl.ds(o, _K)], didx.at[sl], si.at[sl])
            pltpu.async_copy(w_hbm.at[pl.ds(o, _K)], wb.at[sl], si.at[sl])

        def start_gather(ci, sl):
            pltpu.make_async_copy(src_hbm.at[pl.ds(0, _K)], sidx.at[sl], si.at[sl]).wait()
            pltpu.make_async_copy(dst_hbm.at[pl.ds(0, _K)], didx.at[sl], si.at[sl]).wait()
            pltpu.make_async_copy(w_hbm.at[pl.ds(0, _K)], wb.at[sl], si.at[sl]).wait()
            if col_split:
                for j in range(_K // _L):
                    sv = sidx[sl, pl.ds(j * _L, _L)]
                    sidx[sl, pl.ds(j * _L, _L)] = sv + coff
                it = lax.broadcasted_iota(jnp.int32, (_L,), 0)
                cv = jnp.where(it >= (3 * _L - _K), coff, 0)
                sv = sidx[sl, pl.ds(_K - _L, _L)]
                sidx[sl, pl.ds(_K - _L, _L)] = sv + cv
            pltpu.async_copy(h_hbm.at[sidx.at[sl]],
                             rows.at[pl.ds(sl * _K, _K)], sg.at[sl])

        def finish(ci, sl):
            pltpu.make_async_copy(h_hbm.at[sidx.at[sl]],
                                  rows.at[pl.ds(sl * _K, _K)], sg.at[sl]).wait()

            @pl.loop(0, _K // 4)
            def _(i):
                i0 = 4 * i
                r = sl * _K + i0
                wv = [plsc.load_gather(wb.at[sl],
                                       [jnp.full((_L,), i0 + u, jnp.int32)])
                      for u in range(4)]
                for j in range(128 // _L):
                    for u in range(4):
                        rows[r + u, pl.ds(j * _L, _L)] = (
                            rows[r + u, pl.ds(j * _L, _L)] * wv[u])

            pltpu.async_copy(rows.at[pl.ds(sl * _K, _K)],
                             acc.at[didx.at[sl]], ss.at[sl], add=True)

        def retire(sl):
            pltpu.make_async_copy(rows.at[pl.ds(sl * _K, _K)],
                                  acc.at[didx.at[sl]], ss.at[sl]).wait()

        issue_idx(0, 0)
        issue_idx(1, 1)

        @pl.loop(0, _ZR)
        def _(r):
            for j in range(128 // _L):
                zbuf[r, pl.ds(j * _L, _L)] = zeros

        rbase = s * _RPT
        for t in range(_RPT // _ZR):
            pltpu.async_copy(zbuf, acc.at[pl.ds(rbase + t * _ZR, _ZR)],
                             ss.at[0])
        for t in range(_RPT // _ZR):
            pltpu.make_async_copy(zbuf, acc.at[pl.ds(rbase, _ZR)],
                                  ss.at[0]).wait()
        plsc.subcore_barrier()

        start_gather(0, 0)
        for cp in range(_SLOTS):
            if cp >= 3:
                retire((cp + 2) % _SLOTS)
            issue_idx(cp + 2, (cp + 2) % _SLOTS)
            start_gather(cp + 1, (cp + 1) % _SLOTS)
            finish(cp, cp)

        @pl.loop(1, n_blk - 1)
        def _(blk):
            cb = blk * _SLOTS
            for r in range(_SLOTS):
                retire((r + 2) % _SLOTS)
                issue_idx(cb + r + 2, (r + 2) % _SLOTS)
                start_gather(cb + r + 1, (r + 1) % _SLOTS)
                finish(cb + r, r)

        for cp in range(n_ch - _SLOTS, n_ch):
            r = cp % _SLOTS
            retire((r + 2) % _SLOTS)
            if cp + 2 < n_ch:
                issue_idx(cp + 2, (r + 2) % _SLOTS)
            if cp + 1 < n_ch:
                start_gather(cp + 1, (r + 1) % _SLOTS)
            finish(cp, r)
        for sl in (2, 3, 4):
            retire(sl)

        plsc.subcore_barrier()
        coff_out = c * _NP
        for t in range(_RPT // _ZR):
            rr = rbase + t * _ZR
            pltpu.async_copy(acc.at[pl.ds(rr, _ZR)],
                             out_hbm.at[pl.ds(coff_out + rr, _ZR)], ss.at[0])
        for t in range(_RPT // _ZR):
            pltpu.make_async_copy(acc.at[pl.ds(rbase, _ZR)],
                                  out_hbm.at[pl.ds(coff_out + rbase, _ZR)],
                                  ss.at[0]).wait()

    return k


def _sc_agg_cols(h_flat, src, dst, w):
    return _make_sc_agg(True)(h_flat, src, dst, w)


def _sc_agg_edges(h3, src, dst, w):
    return _make_sc_agg(False)(h3, src, dst, w)


def _tc_dinv(degp):
    def body(deg_ref, o_ref):
        deg = jnp.sum(deg_ref[...], axis=0) + 1.0
        o_ref[...] = jnp.where(deg > 0.0,
                               lax.rsqrt(jnp.maximum(deg, 1e-12)), 0.0)

    return pl.pallas_call(
        body,
        out_shape=jax.ShapeDtypeStruct((128, 80), jnp.float32),
    )(degp)


def _tc_matmul(x, W):
    M, K = x.shape
    Nw = W.shape[1]

    def body(x_ref, w_ref, o_ref):
        o_ref[...] = jnp.dot(x_ref[...], w_ref[...],
                             preferred_element_type=jnp.float32)

    return pl.pallas_call(
        body,
        grid=(M // _R,),
        in_specs=[
            pl.BlockSpec((_R, K), lambda i: (i, 0)),
            pl.BlockSpec((K, Nw), lambda i: (0, 0)),
        ],
        out_specs=pl.BlockSpec((_R, Nw), lambda i: (i, 0)),
        out_shape=jax.ShapeDtypeStruct((M, Nw), jnp.float32),
    )(x, W)


def _tc_scale_split(g, dinv):
    def body(g_ref, dinv_ref, o_ref):
        h = g_ref[...] * dinv_ref[...]
        o_ref[0] = h[:, :128]
        o_ref[1] = h[:, 128:]

    return pl.pallas_call(
        body,
        grid=(_NP // _R,),
        in_specs=[
            pl.BlockSpec((_R, 256), lambda i: (i, 0)),
            pl.BlockSpec((_R, 1), lambda i: (i, 0)),
        ],
        out_specs=pl.BlockSpec((2, _R, 128), lambda i: (0, i, 0)),
        out_shape=jax.ShapeDtypeStruct((2, _NP, 128), jnp.float32),
    )(g, dinv)


def _tc_layer(acc, hs, dinv, b, W):
    Nw = W.shape[1]

    def body(acc_ref, hs_ref, dinv_ref, b_ref, w_ref, o_ref):
        dinv = dinv_ref[...]
        bb = b_ref[...]
        t0 = jnp.maximum(dinv * (acc_ref[0] + hs_ref[0]) + bb[:, :128], 0.0)
        t1 = jnp.maximum(dinv * (acc_ref[1] + hs_ref[1]) + bb[:, 128:], 0.0)
        t = jnp.concatenate([t0, t1], axis=1)
        g = jnp.dot(t, w_ref[...], preferred_element_type=jnp.float32)
        h = g * dinv
        if Nw == 256:
            o_ref[0] = h[:, :128]
            o_ref[1] = h[:, 128:]
        else:
            o_ref[...] = h

    if Nw == 256:
        out_spec = pl.BlockSpec((2, _R, 128), lambda i: (0, i, 0))
        out_shape = jax.ShapeDtypeStruct((2, _NP, 128), jnp.float32)
    else:
        out_spec = pl.BlockSpec((_R, Nw), lambda i: (i, 0))
        out_shape = jax.ShapeDtypeStruct((_NP, Nw), jnp.float32)

    return pl.pallas_call(
        body,
        grid=(_NP // _R,),
        in_specs=[
            pl.BlockSpec((2, _R, 128), lambda i: (0, i, 0)),
            pl.BlockSpec((2, _R, 128), lambda i: (0, i, 0)),
            pl.BlockSpec((_R, 1), lambda i: (i, 0)),
            pl.BlockSpec((1, 256), lambda i: (0, 0)),
            pl.BlockSpec(W.shape, lambda i: (0, 0)),
        ],
        out_specs=out_spec,
        out_shape=out_shape,
    )(acc, hs, dinv, b, W)


def _tc_final(accp, h3, dinv, b3):
    def body(acc_ref, h_ref, dinv_ref, b_ref, o_ref):
        sagg = acc_ref[0] + acc_ref[1] + h_ref[...]
        z = (dinv_ref[...] * sagg + b_ref[...])[:, :40]
        m = jnp.max(z, axis=1, keepdims=True)
        ze = z - m
        lse = jnp.log(jnp.sum(jnp.exp(ze), axis=1, keepdims=True))
        o_ref[...] = ze - lse

    return pl.pallas_call(
        body,
        grid=(_NP // _R,),
        in_specs=[
            pl.BlockSpec((2, _R, 128), lambda i: (0, i, 0)),
            pl.BlockSpec((_R, 128), lambda i: (i, 0)),
            pl.BlockSpec((_R, 1), lambda i: (i, 0)),
            pl.BlockSpec((1, 128), lambda i: (0, 0)),
        ],
        out_specs=pl.BlockSpec((_R, 40), lambda i: (i, 0)),
        out_shape=jax.ShapeDtypeStruct((_NP, 40), jnp.float32),
    )(accp, h3, dinv, b3)


def kernel(x, edge_index, edge_attr, W1, b1, W2, b2, W3, b3):
    src = edge_index[0].astype(jnp.int32)
    dst = edge_index[1].astype(jnp.int32)
    w = edge_attr.astype(jnp.float32)
    xp = jnp.pad(x, ((0, _NP - _N), (0, 0)))

    degp = _sc_degree(dst, w)
    g1 = _tc_matmul(xp, W1)
    dinv = _tc_dinv(degp.reshape(_NW, 128, 80)).reshape(_NP, 1)

    h1 = _tc_scale_split(g1, dinv)
    acc1 = _sc_agg_cols(h1.reshape(2 * _NP, 128), src, dst, w)
    h2 = _tc_layer(acc1.reshape(2, _NP, 128), h1, dinv,
                   b1.reshape(1, 256), W2)

    acc2 = _sc_agg_cols(h2.reshape(2 * _NP, 128), src, dst, w)
    W3p = jnp.pad(W3, ((0, 0), (0, 88)))
    h3 = _tc_layer(acc2.reshape(2, _NP, 128), h2, dinv,
                   b2.reshape(1, 256), W3p)

    acc3 = _sc_agg_edges(h3, src, dst, w)
    b3p = jnp.pad(b3, (0, 88)).reshape(1, 128)
    out = _tc_final(acc3.reshape(2, _NP, 128), h3, dinv, b3p)
    return out[:_N]
